# baseline (speedup 1.0000x reference)
import jax
import jax.numpy as jnp
from jax import lax
from jax.experimental import pallas as pl
from jax.experimental.pallas import tpu as pltpu

_N = 10000
_D = 128
_NS = 2
_RS = _N // _NS
_BMS = 40
_MB = _RS // _BMS
_FB = 500


def _fused_body(x_ref, w_ref, g_ref, be_ref, *refs):
    am_refs = refs[:_NS]
    out_ref = refs[_NS]
    xw_ref, s1_ref, s2_ref = refs[_NS + 1:]
    i = pl.program_id(0)

    @pl.when(i == 0)
    def _init():
        xw_ref[...] = lax.dot_general(
            x_ref[...], w_ref[...],
            dimension_numbers=(((1,), (1,)), ((), ())),
            precision=lax.Precision.HIGHEST,
            preferred_element_type=jnp.float32,
        )
        s1_ref[...] = jnp.zeros_like(s1_ref)
        s2_ref[...] = jnp.zeros_like(s2_ref)

    s1 = s1_ref[...]
    s2 = s2_ref[...]
    for s in range(_NS):
        h = lax.dot_general(
            am_refs[s][...], xw_ref[...],
            dimension_numbers=(((1,), (0,)), ((), ())),
            precision=lax.Precision.DEFAULT,
            preferred_element_type=jnp.float32,
        )
        out_ref[pl.ds(s * _RS + i * _BMS, _BMS), :] = h
        s1 = s1 + jnp.sum(h, axis=0, keepdims=True)
        s2 = s2 + jnp.sum(h * h, axis=0, keepdims=True)
    s1_ref[...] = s1
    s2_ref[...] = s2

    @pl.when(i == _MB - 1)
    def _finalize():
        inv_n = jnp.float32(1.0 / _N)
        mean = s1_ref[...] * inv_n
        var = s2_ref[...] * inv_n - mean * mean
        scale = g_ref[...] * lax.rsqrt(var + 1e-5)
        shift = be_ref[...] - mean * scale

        def _norm_chunk(j, carry):
            rows = pl.ds(j * _FB, _FB)
            out_ref[rows, :] = jnp.maximum(out_ref[rows, :] * scale + shift, 0.0)
            return carry

        lax.fori_loop(0, _N // _FB, _norm_chunk, 0)


def kernel(x, am, W, b, gamma, beta):
    del b
    g2 = gamma.reshape(1, _D)
    be2 = beta.reshape(1, _D)

    def _am_spec(s):
        return pl.BlockSpec((_BMS, _N), lambda i, s=s: (s * _MB + i, 0))

    return pl.pallas_call(
        _fused_body,
        grid=(_MB,),
        in_specs=[
            pl.BlockSpec((_N, _D), lambda i: (0, 0)),
            pl.BlockSpec((_D, _D), lambda i: (0, 0)),
            pl.BlockSpec((1, _D), lambda i: (0, 0)),
            pl.BlockSpec((1, _D), lambda i: (0, 0)),
        ] + [_am_spec(s) for s in range(_NS)],
        out_specs=pl.BlockSpec((_N, _D), lambda i: (0, 0)),
        out_shape=jax.ShapeDtypeStruct((_N, _D), jnp.float32),
        scratch_shapes=[
            pltpu.VMEM((_N, _D), jnp.float32),
            pltpu.VMEM((1, _D), jnp.float32),
            pltpu.VMEM((1, _D), jnp.float32),
        ],
    )(x, W, g2, be2, *([am] * _NS))


# speedup vs baseline: 1.4076x; 1.4076x over previous
import jax
import jax.numpy as jnp
from jax import lax
from jax.experimental import pallas as pl
from jax.experimental.pallas import tpu as pltpu

_N = 10000
_D = 128
_BM = 400
_MB = _N // _BM
_FB = 500


def _xw_body(x_ref, w_ref, xw_ref):
    xw_ref[...] = lax.dot_general(
        x_ref[...], w_ref[...],
        dimension_numbers=(((1,), (1,)), ((), ())),
        precision=lax.Precision.HIGHEST,
        preferred_element_type=jnp.float32,
    )


def _main_body(xw_ref, g_ref, be_ref, am_ref, out_ref, s1_ref, s2_ref):
    i = pl.program_id(0)

    @pl.when(i == 0)
    def _init():
        s1_ref[...] = jnp.zeros_like(s1_ref)
        s2_ref[...] = jnp.zeros_like(s2_ref)

    h = lax.dot_general(
        am_ref[...], xw_ref[...],
        dimension_numbers=(((1,), (0,)), ((), ())),
        precision=lax.Precision.DEFAULT,
        preferred_element_type=jnp.float32,
    )
    out_ref[pl.ds(i * _BM, _BM), :] = h
    s1_ref[...] += jnp.sum(h, axis=0, keepdims=True)
    s2_ref[...] += jnp.sum(h * h, axis=0, keepdims=True)

    @pl.when(i == _MB - 1)
    def _finalize():
        inv_n = jnp.float32(1.0 / _N)
        mean = s1_ref[...] * inv_n
        var = s2_ref[...] * inv_n - mean * mean
        scale = g_ref[...] * lax.rsqrt(var + 1e-5)
        shift = be_ref[...] - mean * scale

        def _norm_chunk(j, carry):
            rows = pl.ds(j * _FB, _FB)
            out_ref[rows, :] = jnp.maximum(out_ref[rows, :] * scale + shift, 0.0)
            return carry

        lax.fori_loop(0, _N // _FB, _norm_chunk, 0)


def kernel(x, am, W, b, gamma, beta):
    del b
    xw = pl.pallas_call(
        _xw_body,
        out_shape=jax.ShapeDtypeStruct((_N, _D), jnp.float32),
    )(x, W)

    g2 = gamma.reshape(1, _D)
    be2 = beta.reshape(1, _D)
    return pl.pallas_call(
        _main_body,
        grid=(_MB,),
        in_specs=[
            pl.BlockSpec((_N, _D), lambda i: (0, 0)),
            pl.BlockSpec((1, _D), lambda i: (0, 0)),
            pl.BlockSpec((1, _D), lambda i: (0, 0)),
            pl.BlockSpec((_BM, _N), lambda i: (i, 0)),
        ],
        out_specs=pl.BlockSpec((_N, _D), lambda i: (0, 0)),
        out_shape=jax.ShapeDtypeStruct((_N, _D), jnp.float32),
        scratch_shapes=[
            pltpu.VMEM((1, _D), jnp.float32),
            pltpu.VMEM((1, _D), jnp.float32),
        ],
    )(xw, g2, be2, am)


# speedup vs baseline: 1.4669x; 1.0422x over previous
import jax
import jax.numpy as jnp
from jax import lax
from jax.experimental import pallas as pl
from jax.experimental.pallas import tpu as pltpu

_N = 10000
_D = 128
_BM = 200
_MB = _N // _BM
_FB = 500


def _fused_body(x_ref, w_ref, g_ref, be_ref, am_ref, out_ref, xw_ref, s1_ref, s2_ref):
    i = pl.program_id(0)

    @pl.when(i == 0)
    def _init():
        xw_ref[...] = lax.dot_general(
            x_ref[...], w_ref[...],
            dimension_numbers=(((1,), (1,)), ((), ())),
            precision=lax.Precision.HIGHEST,
            preferred_element_type=jnp.float32,
        )
        s1_ref[...] = jnp.zeros_like(s1_ref)
        s2_ref[...] = jnp.zeros_like(s2_ref)

    h = lax.dot_general(
        am_ref[...], xw_ref[...],
        dimension_numbers=(((1,), (0,)), ((), ())),
        precision=lax.Precision.DEFAULT,
        preferred_element_type=jnp.float32,
    )
    out_ref[pl.ds(i * _BM, _BM), :] = h
    s1_ref[...] += jnp.sum(h, axis=0, keepdims=True)
    s2_ref[...] += jnp.sum(h * h, axis=0, keepdims=True)

    @pl.when(i == _MB - 1)
    def _finalize():
        inv_n = jnp.float32(1.0 / _N)
        mean = s1_ref[...] * inv_n
        var = s2_ref[...] * inv_n - mean * mean
        scale = g_ref[...] * lax.rsqrt(var + 1e-5)
        shift = be_ref[...] - mean * scale

        def _norm_chunk(j, carry):
            rows = pl.ds(j * _FB, _FB)
            out_ref[rows, :] = jnp.maximum(out_ref[rows, :] * scale + shift, 0.0)
            return carry

        lax.fori_loop(0, _N // _FB, _norm_chunk, 0)


def kernel(x, am, W, b, gamma, beta):
    del b
    g2 = gamma.reshape(1, _D)
    be2 = beta.reshape(1, _D)
    return pl.pallas_call(
        _fused_body,
        grid=(_MB,),
        in_specs=[
            pl.BlockSpec((_N, _D), lambda i: (0, 0)),
            pl.BlockSpec((_D, _D), lambda i: (0, 0)),
            pl.BlockSpec((1, _D), lambda i: (0, 0)),
            pl.BlockSpec((1, _D), lambda i: (0, 0)),
            pl.BlockSpec((_BM, _N), lambda i: (i, 0)),
        ],
        out_specs=pl.BlockSpec((_N, _D), lambda i: (0, 0)),
        out_shape=jax.ShapeDtypeStruct((_N, _D), jnp.float32),
        scratch_shapes=[
            pltpu.VMEM((_N, _D), jnp.float32),
            pltpu.VMEM((1, _D), jnp.float32),
            pltpu.VMEM((1, _D), jnp.float32),
        ],
    )(x, W, g2, be2, am)
